# Initial kernel scaffold; baseline (speedup 1.0000x reference)
import jax
import jax.numpy as jnp
from jax.experimental import pallas as pl


def kernel(tensor, rowperm, colperm):
    raise NotImplementedError("write your pallas kernel here")


# speedup vs baseline: 1.0816x; 1.0816x over previous
import functools

import jax
import jax.numpy as jnp
from jax import lax
from jax.experimental import pallas as pl
from jax.experimental.pallas import tpu as pltpu
from jax.experimental.pallas import tpu_sc as plsc

NC = 2
NS = 16
L = 16
NW = NC * NS

R = 8


def _permute(flat, idx, cp):
    N, COL = flat.shape
    rows_per_w = N // NW
    nchunk = rows_per_w // R
    ngrp = COL // L

    mesh = plsc.VectorSubcoreMesh(
        core_axis_name="c", subcore_axis_name="s", num_cores=NC, num_subcores=NS
    )

    @functools.partial(
        pl.kernel,
        mesh=mesh,
        out_type=jax.ShapeDtypeStruct((N, COL), jnp.float32),
        scratch_types=[
            pltpu.VMEM((COL,), jnp.int32),
            pltpu.VMEM((rows_per_w,), jnp.int32),
            pltpu.VMEM((R, COL), jnp.float32),
            pltpu.VMEM((R, COL), jnp.float32),
            pltpu.SemaphoreType.DMA,
        ],
        compiler_params=pltpu.CompilerParams(
            use_tc_tiling_on_sc=False, needs_layout_passes=False
        ),
    )
    def body(flat_hbm, idx_hbm, cp_hbm, out_hbm, cp_v, idx_v, inb, outb, sem):
        wid = lax.axis_index("s") * NC + lax.axis_index("c")
        base = wid * rows_per_w
        pltpu.sync_copy(cp_hbm, cp_v)
        pltpu.sync_copy(idx_hbm.at[pl.ds(base, rows_per_w)], idx_v)

        def chunk(ci, carry):
            pltpu.async_copy(
                flat_hbm.at[idx_v.at[pl.ds(ci * R, R)]], inb, sem
            ).wait()

            def grp(j, c):
                cpj = cp_v[pl.ds(j * L, L)]
                for r in range(R):
                    rr = jnp.full((L,), r, jnp.int32)
                    outb[r, pl.ds(j * L, L)] = plsc.load_gather(inb, [rr, cpj])
                return c

            lax.fori_loop(0, ngrp, grp, 0)
            pltpu.sync_copy(outb, out_hbm.at[pl.ds(base + ci * R, R)])
            return carry

        lax.fori_loop(0, nchunk, chunk, 0)

    return body(flat, idx, cp)


def kernel(tensor, rowperm, colperm):
    B, ROW, COL = tensor.shape
    N = B * ROW
    flat = tensor.reshape(N, COL)
    idx = (
        rowperm.astype(jnp.int32).reshape(1, ROW)
        + (jnp.arange(B, dtype=jnp.int32) * ROW).reshape(B, 1)
    ).reshape(N)
    out = _permute(flat, idx, colperm.astype(jnp.int32))
    return out.reshape(B, ROW, COL)


# speedup vs baseline: 1.3543x; 1.2521x over previous
import functools

import jax
import jax.numpy as jnp
from jax import lax
from jax.experimental import pallas as pl
from jax.experimental.pallas import tpu as pltpu
from jax.experimental.pallas import tpu_sc as plsc

NC = 2
NS = 16
L = 16
NW = NC * NS

R = 8


def _permute(flat, idx, cp):
    N, COL = flat.shape
    rows_per_w = N // NW
    nchunk = rows_per_w // R
    ngrp = COL // L
    assert nchunk % 2 == 0

    mesh = plsc.VectorSubcoreMesh(
        core_axis_name="c", subcore_axis_name="s", num_cores=NC, num_subcores=NS
    )

    @functools.partial(
        pl.kernel,
        mesh=mesh,
        out_type=jax.ShapeDtypeStruct((N, COL), jnp.float32),
        scratch_types=[
            pltpu.VMEM((COL,), jnp.int32),
            pltpu.VMEM((rows_per_w,), jnp.int32),
            pltpu.VMEM((2, R, COL), jnp.float32),
            pltpu.VMEM((2, R, COL), jnp.float32),
            pltpu.SemaphoreType.DMA((2,)),
            pltpu.SemaphoreType.DMA((2,)),
        ],
        compiler_params=pltpu.CompilerParams(
            use_tc_tiling_on_sc=False, needs_layout_passes=False
        ),
    )
    def body(flat_hbm, idx_hbm, cp_hbm, out_hbm, cp_v, idx_v, inb, outb, gsem, ssem):
        wid = lax.axis_index("s") * NC + lax.axis_index("c")
        base = wid * rows_per_w
        pltpu.sync_copy(cp_hbm, cp_v)
        pltpu.sync_copy(idx_hbm.at[pl.ds(base, rows_per_w)], idx_v)

        def start_gather(ci, b):
            pltpu.async_copy(
                flat_hbm.at[idx_v.at[pl.ds(ci * R, R)]], inb.at[b], gsem.at[b]
            )

        def wait_gather(ci, b):
            pltpu.make_async_copy(
                flat_hbm.at[idx_v.at[pl.ds(ci * R, R)]], inb.at[b], gsem.at[b]
            ).wait()

        def start_scatter(ci, b):
            pltpu.async_copy(
                outb.at[b], out_hbm.at[pl.ds(base + ci * R, R)], ssem.at[b]
            )

        def wait_scatter(ci, b):
            pltpu.make_async_copy(
                outb.at[b], out_hbm.at[pl.ds(base + ci * R, R)], ssem.at[b]
            ).wait()

        rfull = [jnp.full((L,), r, jnp.int32) for r in range(R)]

        def compute(b):
            def grp(j, c):
                cpj = cp_v[pl.ds(j * L, L)]
                for r in range(R):
                    outb[b, r, pl.ds(j * L, L)] = plsc.load_gather(
                        inb.at[b], [rfull[r], cpj]
                    )
                return c

            lax.fori_loop(0, ngrp, grp, 0)

        start_gather(0, 0)
        start_gather(1, 1)

        def step(t, c):
            for b in range(2):
                ci = 2 * t + b

                @pl.when(t > 0)
                def _():
                    wait_scatter(ci - 2, b)

                wait_gather(ci, b)
                compute(b)
                start_scatter(ci, b)

                @pl.when(ci + 2 < nchunk)
                def _():
                    start_gather(ci + 2, b)

            return c

        lax.fori_loop(0, nchunk // 2, step, 0)
        wait_scatter(nchunk - 2, 0)
        wait_scatter(nchunk - 1, 1)

    return body(flat, idx, cp)


def kernel(tensor, rowperm, colperm):
    B, ROW, COL = tensor.shape
    N = B * ROW
    flat = tensor.reshape(N, COL)
    idx = (
        rowperm.astype(jnp.int32).reshape(1, ROW)
        + (jnp.arange(B, dtype=jnp.int32) * ROW).reshape(B, 1)
    ).reshape(N)
    out = _permute(flat, idx, colperm.astype(jnp.int32))
    return out.reshape(B, ROW, COL)


# speedup vs baseline: 2.1272x; 1.5707x over previous
import functools

import jax
import jax.numpy as jnp
from jax import lax
from jax.experimental import pallas as pl
from jax.experimental.pallas import tpu as pltpu
from jax.experimental.pallas import tpu_sc as plsc

NC = 2
NS = 16
L = 16
NW = NC * NS

R = 8


def _permute(flat, idx, cp):
    N, COL = flat.shape
    rows_per_w = N // NW
    nchunk = rows_per_w // R
    ngrp = COL // L
    assert nchunk % 2 == 0

    mesh = plsc.VectorSubcoreMesh(
        core_axis_name="c", subcore_axis_name="s", num_cores=NC, num_subcores=NS
    )

    @functools.partial(
        pl.kernel,
        mesh=mesh,
        out_type=jax.ShapeDtypeStruct((N, COL), jnp.float32),
        scratch_types=[
            pltpu.VMEM((COL,), jnp.int32),
            pltpu.VMEM((rows_per_w,), jnp.int32),
            pltpu.VMEM((2, R, COL), jnp.float32),
            pltpu.VMEM((2, R, COL), jnp.float32),
            pltpu.SemaphoreType.DMA((2,)),
            pltpu.SemaphoreType.DMA((2,)),
        ],
        compiler_params=pltpu.CompilerParams(
            use_tc_tiling_on_sc=False, needs_layout_passes=False
        ),
    )
    def body(flat_hbm, idx_hbm, cp_hbm, out_hbm, cp_v, idx_v, inb, outb, gsem, ssem):
        wid = lax.axis_index("s") * NC + lax.axis_index("c")
        base = wid * rows_per_w
        pltpu.sync_copy(cp_hbm, cp_v)
        pltpu.sync_copy(idx_hbm.at[pl.ds(base, rows_per_w)], idx_v)

        def start_gather(ci, b):
            pltpu.async_copy(
                flat_hbm.at[idx_v.at[pl.ds(ci * R, R)]], inb.at[b], gsem.at[b]
            )

        def wait_gather(ci, b):
            pltpu.make_async_copy(
                flat_hbm.at[idx_v.at[pl.ds(ci * R, R)]], inb.at[b], gsem.at[b]
            ).wait()

        def start_scatter(ci, b):
            pltpu.async_copy(
                outb.at[b], out_hbm.at[pl.ds(base + ci * R, R)], ssem.at[b]
            )

        def wait_scatter(ci, b):
            pltpu.make_async_copy(
                outb.at[b], out_hbm.at[pl.ds(base + ci * R, R)], ssem.at[b]
            ).wait()

        rfull = [jnp.full((L,), r, jnp.int32) for r in range(R)]

        def compute(b):
            @plsc.parallel_loop(0, ngrp, unroll=2)
            def _grp(j):
                cpj = cp_v[pl.ds(j * L, L)]
                for r in range(R):
                    v = plsc.load_gather(inb.at[b], [rfull[r], cpj])
                    outb.at[b, r].at[pl.ds(j * L, L)].set(v)

        start_gather(0, 0)
        start_gather(1, 1)

        def step(t, c):
            for b in range(2):
                ci = 2 * t + b

                @pl.when(t > 0)
                def _():
                    wait_scatter(ci - 2, b)

                wait_gather(ci, b)
                compute(b)
                start_scatter(ci, b)

                @pl.when(ci + 2 < nchunk)
                def _():
                    start_gather(ci + 2, b)

            return c

        lax.fori_loop(0, nchunk // 2, step, 0)
        wait_scatter(nchunk - 2, 0)
        wait_scatter(nchunk - 1, 1)

    return body(flat, idx, cp)


def kernel(tensor, rowperm, colperm):
    B, ROW, COL = tensor.shape
    N = B * ROW
    flat = tensor.reshape(N, COL)
    idx = (
        rowperm.astype(jnp.int32).reshape(1, ROW)
        + (jnp.arange(B, dtype=jnp.int32) * ROW).reshape(B, 1)
    ).reshape(N)
    out = _permute(flat, idx, colperm.astype(jnp.int32))
    return out.reshape(B, ROW, COL)


# speedup vs baseline: 2.1442x; 1.0080x over previous
import functools
import math

import jax
import jax.numpy as jnp
from jax import lax
from jax.experimental import pallas as pl
from jax.experimental.pallas import tpu as pltpu
from jax.experimental.pallas import tpu_sc as plsc

NC = 2
NS = 16
L = 16
NW = NC * NS

R = 8
NBUF_IN = 4
NBUF_OUT = 2


def _permute(flat, idx, cp):
    N, COL = flat.shape
    rows_per_w = N // NW
    nchunk = rows_per_w // R
    ngrp = COL // L
    assert nchunk % NBUF_IN == 0 and nchunk % NBUF_OUT == 0
    assert nchunk >= NBUF_IN

    mesh = plsc.VectorSubcoreMesh(
        core_axis_name="c", subcore_axis_name="s", num_cores=NC, num_subcores=NS
    )

    @functools.partial(
        pl.kernel,
        mesh=mesh,
        out_type=jax.ShapeDtypeStruct((N, COL), jnp.float32),
        scratch_types=[
            pltpu.VMEM((COL,), jnp.int32),
            pltpu.VMEM((rows_per_w,), jnp.int32),
            pltpu.VMEM((NBUF_IN, R, COL), jnp.float32),
            pltpu.VMEM((NBUF_OUT, R, COL), jnp.float32),
            pltpu.SemaphoreType.DMA((NBUF_IN,)),
            pltpu.SemaphoreType.DMA((NBUF_OUT,)),
        ],
        compiler_params=pltpu.CompilerParams(
            use_tc_tiling_on_sc=False, needs_layout_passes=False
        ),
    )
    def body(flat_hbm, idx_hbm, cp_hbm, out_hbm, cp_v, idx_v, inb, outb, gsem, ssem):
        wid = lax.axis_index("s") * NC + lax.axis_index("c")
        base = wid * rows_per_w
        pltpu.sync_copy(cp_hbm, cp_v)
        pltpu.sync_copy(idx_hbm.at[pl.ds(base, rows_per_w)], idx_v)

        def start_gather(ci, b):
            pltpu.async_copy(
                flat_hbm.at[idx_v.at[pl.ds(ci * R, R)]], inb.at[b], gsem.at[b]
            )

        def wait_gather(ci, b):
            pltpu.make_async_copy(
                flat_hbm.at[idx_v.at[pl.ds(ci * R, R)]], inb.at[b], gsem.at[b]
            ).wait()

        def start_scatter(ci, b):
            pltpu.async_copy(
                outb.at[b], out_hbm.at[pl.ds(base + ci * R, R)], ssem.at[b]
            )

        def wait_scatter(ci, b):
            pltpu.make_async_copy(
                outb.at[b], out_hbm.at[pl.ds(base + ci * R, R)], ssem.at[b]
            ).wait()

        rfull = [jnp.full((L,), r, jnp.int32) for r in range(R)]

        def compute(bi, bo):
            @plsc.parallel_loop(0, ngrp, unroll=2)
            def _grp(j):
                cpj = cp_v[pl.ds(j * L, L)]
                for r in range(R):
                    v = plsc.load_gather(inb.at[bi], [rfull[r], cpj])
                    outb.at[bo, r].at[pl.ds(j * L, L)].set(v)

        for b in range(NBUF_IN):
            start_gather(b, b)

        nper = math.lcm(NBUF_IN, NBUF_OUT)

        def step(t, c):
            for k in range(nper):
                ci = nper * t + k
                bi = ci % NBUF_IN
                bo = ci % NBUF_OUT

                @pl.when(ci >= NBUF_OUT)
                def _():
                    wait_scatter(ci - NBUF_OUT, bo)

                wait_gather(ci, bi)
                compute(bi, bo)
                start_scatter(ci, bo)

                @pl.when(ci + NBUF_IN < nchunk)
                def _():
                    start_gather(ci + NBUF_IN, bi)

            return c

        lax.fori_loop(0, nchunk // nper, step, 0)
        for b in range(NBUF_OUT):
            wait_scatter(nchunk - NBUF_OUT + b, (nchunk - NBUF_OUT + b) % NBUF_OUT)

    return body(flat, idx, cp)


def kernel(tensor, rowperm, colperm):
    B, ROW, COL = tensor.shape
    N = B * ROW
    flat = tensor.reshape(N, COL)
    idx = (
        rowperm.astype(jnp.int32).reshape(1, ROW)
        + (jnp.arange(B, dtype=jnp.int32) * ROW).reshape(B, 1)
    ).reshape(N)
    out = _permute(flat, idx, colperm.astype(jnp.int32))
    return out.reshape(B, ROW, COL)
